# baseline (device time: 23350 ns/iter reference)
import jax
import jax.numpy as jnp
from jax import lax
from jax.experimental import pallas as pl
from jax.experimental.pallas import tpu as pltpu

N_DEV = 16
N_GROUPS = 4
BLKS_PER_GROUP = N_DEV // N_GROUPS
_GELU_C = 0.7978845608028654


def _gelu(y):
    return 0.5 * y * (1.0 + jnp.tanh(_GELU_C * (y + 0.044715 * y * y * y)))


def kernel(x, w_mat):
    m_per, k = x.shape
    _, n = w_mat.shape
    n_per = n // N_DEV
    grp_cols = n // N_GROUPS

    def body(x_ref, w_ref, out_ref, w_slab, y_send, r_buf, load_sems,
             send_sems, recv_sems):
        me = lax.axis_index("i")
        my_quad = lax.div(me, BLKS_PER_GROUP)

        def quad_of(g):
            return lax.rem(my_quad + g, N_GROUPS)

        def start_load(g):
            slot = g % 2
            pltpu.make_async_copy(
                w_ref.at[:, pl.ds(quad_of(g) * grp_cols, grp_cols)],
                w_slab.at[slot],
                load_sems.at[slot],
            ).start()

        def wait_load(g):
            slot = g % 2
            pltpu.make_async_copy(
                w_ref.at[:, pl.ds(quad_of(g) * grp_cols, grp_cols)],
                w_slab.at[slot],
                load_sems.at[slot],
            ).wait()

        start_load(0)
        start_load(1)
        barrier_sem = pltpu.get_barrier_semaphore()
        for s in range(1, N_DEV):
            pl.semaphore_signal(
                barrier_sem,
                inc=1,
                device_id=(lax.rem(me + s, N_DEV),),
                device_id_type=pl.DeviceIdType.MESH,
            )

        for g in range(N_GROUPS):
            wait_load(g)
            base_blk = quad_of(g) * BLKS_PER_GROUP
            yg = _gelu(
                jnp.dot(
                    x_ref[:, :],
                    w_slab[g % 2],
                    preferred_element_type=jnp.float32,
                )
            )
            if g == 0:
                pl.semaphore_wait(barrier_sem, N_DEV - 1)
            for q in range(BLKS_PER_GROUP):
                j = base_blk + q
                blk = yg[:, q * n_per:(q + 1) * n_per]

                @pl.when(j == me)
                def _():
                    out_ref[pl.ds(me * m_per, m_per), :] = blk

                @pl.when(j != me)
                def _():
                    s = lax.rem(j - me + N_DEV, N_DEV)
                    y_send[pl.ds(s - 1, 1), :, :] = blk.astype(jnp.bfloat16)[None]
                    pltpu.make_async_remote_copy(
                        src_ref=y_send.at[s - 1],
                        dst_ref=r_buf.at[s - 1],
                        send_sem=send_sems.at[s - 1],
                        recv_sem=recv_sems.at[s - 1],
                        device_id=(j,),
                        device_id_type=pl.DeviceIdType.MESH,
                    ).start()
            if g + 2 < N_GROUPS:
                start_load(g + 2)

        for s in range(1, N_DEV):
            d = pltpu.make_async_remote_copy(
                src_ref=y_send.at[s - 1],
                dst_ref=r_buf.at[s - 1],
                send_sem=send_sems.at[s - 1],
                recv_sem=recv_sems.at[s - 1],
                device_id=(me,),
                device_id_type=pl.DeviceIdType.MESH,
            )
            d.wait_send()
            d.wait_recv()
            i = lax.rem(me - s + N_DEV, N_DEV)
            out_ref[pl.ds(i * m_per, m_per), :] = r_buf[s - 1, :, :].astype(
                jnp.float32
            )

    return pl.pallas_call(
        body,
        out_shape=jax.ShapeDtypeStruct((N_DEV * m_per, n_per), jnp.float32),
        in_specs=[
            pl.BlockSpec(memory_space=pltpu.VMEM),
            pl.BlockSpec(memory_space=pltpu.MemorySpace.HBM),
        ],
        out_specs=pl.BlockSpec(memory_space=pltpu.VMEM),
        scratch_shapes=[
            pltpu.VMEM((2, k, grp_cols), jnp.float32),
            pltpu.VMEM((N_DEV - 1, m_per, n_per), jnp.bfloat16),
            pltpu.VMEM((N_DEV - 1, m_per, n_per), jnp.bfloat16),
            pltpu.SemaphoreType.DMA((2,)),
            pltpu.SemaphoreType.DMA((N_DEV - 1,)),
            pltpu.SemaphoreType.DMA((N_DEV - 1,)),
        ],
        compiler_params=pltpu.CompilerParams(collective_id=0),
    )(x, w_mat)


# device time: 23272 ns/iter; 1.0034x vs baseline; 1.0034x over previous
import jax
import jax.numpy as jnp
from jax import lax
from jax.experimental import pallas as pl
from jax.experimental.pallas import tpu as pltpu

N_DEV = 16
N_GROUPS = 4
BLKS_PER_GROUP = N_DEV // N_GROUPS
_GELU_C = 0.7978845608028654


def _gelu(y):
    return 0.5 * y * (1.0 + jnp.tanh(_GELU_C * (y + 0.044715 * y * y * y)))


def kernel(x, w_mat):
    m_per, k = x.shape
    _, n = w_mat.shape
    n_per = n // N_DEV
    grp_cols = n // N_GROUPS

    def body(x_ref, w_ref, out_ref, w_slab, y_send, r_buf, load_sems,
             send_sems, recv_sems):
        me = lax.axis_index("i")
        my_quad = lax.div(me, BLKS_PER_GROUP)

        def quad_of(g):
            return lax.rem(my_quad + g, N_GROUPS)

        def start_load(g):
            slot = g % 2
            pltpu.make_async_copy(
                w_ref.at[:, pl.ds(quad_of(g) * grp_cols, grp_cols)],
                w_slab.at[slot],
                load_sems.at[slot],
            ).start()

        def wait_load(g):
            slot = g % 2
            pltpu.make_async_copy(
                w_ref.at[:, pl.ds(quad_of(g) * grp_cols, grp_cols)],
                w_slab.at[slot],
                load_sems.at[slot],
            ).wait()

        start_load(0)
        start_load(1)
        barrier_sem = pltpu.get_barrier_semaphore()
        for s in range(1, N_DEV):
            pl.semaphore_signal(
                barrier_sem,
                inc=1,
                device_id=(lax.rem(me + s, N_DEV),),
                device_id_type=pl.DeviceIdType.MESH,
            )

        for g in range(N_GROUPS):
            wait_load(g)
            base_blk = quad_of(g) * BLKS_PER_GROUP
            yg = _gelu(
                jnp.dot(
                    x_ref[:, :],
                    w_slab[g % 2],
                    preferred_element_type=jnp.float32,
                )
            )
            if g == 0:
                pl.semaphore_wait(barrier_sem, N_DEV - 1)
            for q in range(BLKS_PER_GROUP):
                j = base_blk + q
                blk = yg[:, q * n_per:(q + 1) * n_per]

                @pl.when(j == me)
                def _():
                    out_ref[pl.ds(me * m_per, m_per), :] = blk

                @pl.when(j != me)
                def _():
                    s = lax.rem(j - me + N_DEV, N_DEV)
                    y_send[pl.ds(s - 1, 1), :, :] = blk.astype(jnp.bfloat16)[None]
                    pltpu.make_async_remote_copy(
                        src_ref=y_send.at[s - 1],
                        dst_ref=r_buf.at[s - 1],
                        send_sem=send_sems.at[s - 1],
                        recv_sem=recv_sems.at[s - 1],
                        device_id=(j,),
                        device_id_type=pl.DeviceIdType.MESH,
                    ).start()
            if g + 2 < N_GROUPS:
                start_load(g + 2)

        for s in range(1, N_DEV):
            d = pltpu.make_async_remote_copy(
                src_ref=y_send.at[s - 1],
                dst_ref=r_buf.at[s - 1],
                send_sem=send_sems.at[s - 1],
                recv_sem=recv_sems.at[s - 1],
                device_id=(me,),
                device_id_type=pl.DeviceIdType.MESH,
            )
            d.wait_send()
            d.wait_recv()
            i = lax.rem(me - s + N_DEV, N_DEV)
            out_ref[pl.ds(i * m_per, m_per), :] = r_buf[s - 1, :, :].astype(
                jnp.float32
            )

    return pl.pallas_call(
        body,
        out_shape=jax.ShapeDtypeStruct((N_DEV * m_per, n_per), jnp.float32),
        in_specs=[
            pl.BlockSpec(memory_space=pltpu.VMEM),
            pl.BlockSpec(memory_space=pl.ANY),
        ],
        out_specs=pl.BlockSpec(memory_space=pltpu.VMEM),
        scratch_shapes=[
            pltpu.VMEM((2, k, grp_cols), jnp.float32),
            pltpu.VMEM((N_DEV - 1, m_per, n_per), jnp.bfloat16),
            pltpu.VMEM((N_DEV - 1, m_per, n_per), jnp.bfloat16),
            pltpu.SemaphoreType.DMA((2,)),
            pltpu.SemaphoreType.DMA((N_DEV - 1,)),
            pltpu.SemaphoreType.DMA((N_DEV - 1,)),
        ],
        compiler_params=pltpu.CompilerParams(collective_id=0),
    )(x, w_mat)


# device time: 20214 ns/iter; 1.1551x vs baseline; 1.1513x over previous
import jax
import jax.numpy as jnp
from jax import lax
from jax.experimental import pallas as pl
from jax.experimental.pallas import tpu as pltpu

N_DEV = 16
N_GROUPS = 4
BLKS_PER_GROUP = N_DEV // N_GROUPS
_GELU_C = 0.7978845608028654


def _gelu(y):
    return 0.5 * y * (1.0 + jnp.tanh(_GELU_C * (y + 0.044715 * y * y * y)))


def kernel(x, w_mat):
    m_per, k = x.shape
    _, n = w_mat.shape
    n_per = n // N_DEV
    grp_cols = n // N_GROUPS

    def body(x_ref, w_ref, out_ref, y_send, r_buf, send_sems, recv_sems):
        me = lax.axis_index("i")
        my_quad = lax.div(me, BLKS_PER_GROUP)

        barrier_sem = pltpu.get_barrier_semaphore()
        for s in range(1, N_DEV):
            pl.semaphore_signal(
                barrier_sem,
                inc=1,
                device_id=(lax.rem(me + s, N_DEV),),
                device_id_type=pl.DeviceIdType.MESH,
            )

        for g in range(N_GROUPS):
            quad = lax.rem(my_quad + g, N_GROUPS)
            base_blk = quad * BLKS_PER_GROUP
            yg = _gelu(
                jnp.dot(
                    x_ref[:, :],
                    w_ref[:, pl.ds(base_blk * n_per, grp_cols)],
                    preferred_element_type=jnp.float32,
                )
            )
            if g == 0:
                pl.semaphore_wait(barrier_sem, N_DEV - 1)
            for q in range(BLKS_PER_GROUP):
                j = base_blk + q
                blk = yg[:, q * n_per:(q + 1) * n_per]

                @pl.when(j == me)
                def _():
                    out_ref[pl.ds(me * m_per, m_per), :] = blk

                @pl.when(j != me)
                def _():
                    y_send[pl.ds(j, 1), :, :] = blk.astype(jnp.bfloat16)[None]
                    pltpu.make_async_remote_copy(
                        src_ref=y_send.at[j],
                        dst_ref=r_buf.at[me],
                        send_sem=send_sems.at[j],
                        recv_sem=recv_sems.at[me],
                        device_id=(j,),
                        device_id_type=pl.DeviceIdType.MESH,
                    ).start()

        def wait_dma(sem_arr, idx, ref):
            pltpu.make_async_copy(ref, ref, sem_arr.at[idx]).wait()

        for g in range(N_GROUPS):
            src_quad = lax.rem(my_quad - g + N_GROUPS, N_GROUPS)
            base = src_quad * BLKS_PER_GROUP
            for q in range(BLKS_PER_GROUP):
                i = base + q

                @pl.when(i != me)
                def _():
                    wait_dma(recv_sems, i, r_buf.at[i])

            if g == 0:
                for q in range(BLKS_PER_GROUP):
                    i = base + q

                    @pl.when(i != me)
                    def _():
                        out_ref[pl.ds(i * m_per, m_per), :] = r_buf[
                            pl.ds(i, 1), :, :
                        ][0].astype(jnp.float32)
            else:
                out_ref[pl.ds(base * m_per, BLKS_PER_GROUP * m_per), :] = (
                    r_buf[pl.ds(base, BLKS_PER_GROUP), :, :]
                    .reshape(BLKS_PER_GROUP * m_per, n_per)
                    .astype(jnp.float32)
                )

        for j in range(N_DEV):

            @pl.when(j != me)
            def _():
                wait_dma(send_sems, j, y_send.at[j])

    return pl.pallas_call(
        body,
        out_shape=jax.ShapeDtypeStruct((N_DEV * m_per, n_per), jnp.float32),
        in_specs=[
            pl.BlockSpec(memory_space=pltpu.VMEM),
            pl.BlockSpec(memory_space=pltpu.VMEM),
        ],
        out_specs=pl.BlockSpec(memory_space=pltpu.VMEM),
        scratch_shapes=[
            pltpu.VMEM((N_DEV, m_per, n_per), jnp.bfloat16),
            pltpu.VMEM((N_DEV, m_per, n_per), jnp.bfloat16),
            pltpu.SemaphoreType.DMA((N_DEV,)),
            pltpu.SemaphoreType.DMA((N_DEV,)),
        ],
        compiler_params=pltpu.CompilerParams(collective_id=0),
    )(x, w_mat)
